# baseline (speedup 1.0000x reference)
import functools

import jax
import jax.numpy as jnp
from jax import lax
from jax.experimental import pallas as pl
from jax.experimental.pallas import tpu as pltpu
from jax.experimental.pallas import tpu_sc as plsc

_N = 100000
_F = 512
_E = 51
_EP = 128
_H = 128
_O = 64
_B = 4096
_NEG = 4
_R = 5000


def _dense_body(x1_ref, x2_ref, x3_ref, x4_ref, gt_ref, m_ref, w_ref, b_ref,
                w0_ref, w1_ref, logits_ref, lp_ref, loss_ref, acc_ref):
    i = pl.program_id(0)

    q = _F // 4
    y = (jnp.dot(x1_ref[...], w_ref[0:q], preferred_element_type=jnp.float32)
         + jnp.dot(x2_ref[...], w_ref[q:2 * q], preferred_element_type=jnp.float32)
         + jnp.dot(x3_ref[...], w_ref[2 * q:3 * q], preferred_element_type=jnp.float32)
         + jnp.dot(x4_ref[...], w_ref[3 * q:4 * q], preferred_element_type=jnp.float32)
         + b_ref[...])
    y = jnp.maximum(y, 0.2 * y)
    col = lax.broadcasted_iota(jnp.int32, y.shape, 1)
    y = jnp.where(col < _E, y, 0.0)
    mode = jnp.sqrt(jnp.sum(y * y, axis=-1, keepdims=True))
    lg = y / mode
    logits_ref[...] = lg

    h = jnp.maximum(jnp.dot(lg, w0_ref[...], preferred_element_type=jnp.float32), 0.0)
    lp = jnp.dot(h, w1_ref[...], preferred_element_type=jnp.float32)
    lp_ref[...] = lp

    mx = jnp.max(lp, axis=-1, keepdims=True)
    lse = mx + jnp.log(jnp.sum(jnp.exp(lp - mx), axis=-1, keepdims=True))
    ce = -jnp.sum(gt_ref[...] * (lp - lse), axis=-1, keepdims=True)
    m = m_ref[0]

    @pl.when(i == 0)
    def _():
        acc_ref[0] = 0.0
        acc_ref[1] = 0.0

    acc_ref[0] += jnp.dot(m, ce, preferred_element_type=jnp.float32)[0, 0]
    acc_ref[1] += jnp.sum(m)

    @pl.when(i == pl.num_programs(0) - 1)
    def _():
        w0 = w0_ref[...]
        w1 = w1_ref[...]
        l2 = 0.5 * (jnp.sum(w0 * w0) + jnp.sum(w1 * w1))
        loss_ref[...] = jnp.reshape(acc_ref[0] / acc_ref[1] + 0.0002 * l2, (1, 1))


def _dense_pass(inputs, ground_truth, masks_f, w_pad, b2, w0_pad, out_w1):
    grid = (_N // _R,)
    return pl.pallas_call(
        _dense_body,
        grid=grid,
        in_specs=[
            pl.BlockSpec((_R, _F // 4), lambda i: (i, 0)),
            pl.BlockSpec((_R, _F // 4), lambda i: (i, 1)),
            pl.BlockSpec((_R, _F // 4), lambda i: (i, 2)),
            pl.BlockSpec((_R, _F // 4), lambda i: (i, 3)),
            pl.BlockSpec((_R, _O), lambda i: (i, 0)),
            pl.BlockSpec((1, 1, _R), lambda i: (i, 0, 0)),
            pl.BlockSpec((_F, _EP), lambda i: (0, 0)),
            pl.BlockSpec((1, 1), lambda i: (0, 0)),
            pl.BlockSpec((_EP, _H), lambda i: (0, 0)),
            pl.BlockSpec((_H, _O), lambda i: (0, 0)),
        ],
        out_specs=[
            pl.BlockSpec((_R, _EP), lambda i: (i, 0)),
            pl.BlockSpec((_R, _O), lambda i: (i, 0)),
            pl.BlockSpec((1, 1), lambda i: (0, 0)),
        ],
        out_shape=[
            jax.ShapeDtypeStruct((_N, _EP), jnp.float32),
            jax.ShapeDtypeStruct((_N, _O), jnp.float32),
            jax.ShapeDtypeStruct((1, 1), jnp.float32),
        ],
        scratch_shapes=[pltpu.SMEM((2,), jnp.float32)],
        compiler_params=pltpu.CompilerParams(
            dimension_semantics=("arbitrary",)),
    )(inputs, inputs, inputs, inputs, ground_truth, masks_f, w_pad, b2, w0_pad, out_w1)


_NUM_IDS = _B * (2 + _NEG)
_CHUNK = 128
_EPG = 2 + _NEG
_DOT = 128


def _sc_gather_dots(table, ids):
    info = plsc.get_sparse_core_info()
    nw = info.num_cores * info.num_subcores
    per_w = _NUM_IDS // nw
    e_per_w = per_w // _EPG
    n_ch = per_w // _CHUNK
    mesh = plsc.VectorSubcoreMesh(core_axis_name="c", subcore_axis_name="s")

    @functools.partial(
        pl.kernel,
        mesh=mesh,
        out_type=jax.ShapeDtypeStruct((_B, _DOT), jnp.float32),
        scratch_types=[
            pltpu.VMEM((per_w,), jnp.int32),
            pltpu.VMEM((per_w, _EP), jnp.float32),
            pltpu.VMEM((e_per_w, _DOT), jnp.float32),
            pltpu.SemaphoreType.DMA,
        ],
    )
    def k(table_hbm, idx_hbm, out_hbm, idx_v, rows_v, dots_v, sem):
        wid = lax.axis_index("s") * info.num_cores + lax.axis_index("c")
        base = wid * per_w
        pltpu.sync_copy(idx_hbm.at[pl.ds(base, per_w)], idx_v)
        descs = []
        for c in range(n_ch):
            descs.append(pltpu.async_copy(
                table_hbm.at[idx_v.at[pl.ds(c * _CHUNK, _CHUNK)]],
                rows_v.at[pl.ds(c * _CHUNK, _CHUNK)],
                sem))
        for d in descs:
            d.wait()

        def edge(e, carry):
            r0 = e * _EPG
            hi = [rows_v[r0, pl.ds(c * 16, 16)] for c in range(4)]
            for p in range(1 + _NEG):
                acc = hi[0] * rows_v[r0 + 1 + p, pl.ds(0, 16)]
                for c in range(1, 4):
                    acc += hi[c] * rows_v[r0 + 1 + p, pl.ds(c * 16, 16)]
                dots_v[e, pl.ds(p * 16, 16)] = acc
            return carry

        lax.fori_loop(0, e_per_w, edge, 0)
        pltpu.sync_copy(dots_v, out_hbm.at[pl.ds(wid * e_per_w, e_per_w)])

    return k(table, ids)


def _struct_body(d_ref, out_ref):
    lane = lax.broadcasted_iota(jnp.int32, (80, 8), 0)
    grp = lax.broadcasted_iota(jnp.int32, (80, 8), 1)
    sel = (lane // 16) == grp
    s = jnp.where(sel & (grp == 0), -1.0, jnp.where(sel & (grp < 5), 1.0, 0.0))
    x = jnp.dot(d_ref[:, 0:80], s, preferred_element_type=jnp.float32)
    sp = jnp.log1p(jnp.exp(x))
    wv = jnp.where(grp[0:1, :] == 0, 1.0, jnp.where(grp[0:1, :] < 5, 1.0 / _NEG, 0.0))
    total = jnp.sum(sp * wv) / _B
    out_ref[...] = jnp.reshape(total, (1, 1))


def _struct_tail(dots):
    return pl.pallas_call(
        _struct_body,
        out_shape=jax.ShapeDtypeStruct((1, 1), jnp.float32),
    )(dots)


def kernel(inputs, ground_truth, masks, xi_id, xj_id, neg_xj_id, w, b, out_w0, out_w1):
    w_pad = jnp.pad(w, ((0, 0), (0, _EP - _E)))
    w0_pad = jnp.pad(out_w0, ((0, _EP - _E), (0, 0)))
    b2 = jnp.reshape(b, (1, 1))
    masks_f = masks.astype(jnp.float32).reshape(_N // _R, 1, _R)

    logits, label_pred, label_loss = _dense_pass(
        inputs, ground_truth, masks_f, w_pad, b2, w0_pad, out_w1)

    ids = jnp.concatenate([xi_id, xj_id, neg_xj_id], axis=1).reshape(-1).astype(jnp.int32)

    dots = _sc_gather_dots(logits, ids)
    struct = _struct_tail(dots)

    return struct[0, 0], label_loss[0, 0], label_pred


# speedup vs baseline: 1.0125x; 1.0125x over previous
import functools

import jax
import jax.numpy as jnp
from jax import lax
from jax.experimental import pallas as pl
from jax.experimental.pallas import tpu as pltpu
from jax.experimental.pallas import tpu_sc as plsc

_N = 100000
_F = 512
_E = 51
_EP = 128
_H = 128
_O = 64
_B = 4096
_NEG = 4
_R = 5000


def _dense_body(x1_ref, x2_ref, gt_ref, m_ref, w_ref, b_ref,
                w0_ref, w1_ref, logits_ref, lp_ref, loss_ref, acc_ref):
    i = pl.program_id(0)

    y = (jnp.dot(x1_ref[...], w_ref[0:_F // 2], preferred_element_type=jnp.float32)
         + jnp.dot(x2_ref[...], w_ref[_F // 2:_F], preferred_element_type=jnp.float32)
         + b_ref[...])
    y = jnp.maximum(y, 0.2 * y)
    col = lax.broadcasted_iota(jnp.int32, y.shape, 1)
    y = jnp.where(col < _E, y, 0.0)
    mode = jnp.sqrt(jnp.sum(y * y, axis=-1, keepdims=True))
    lg = y / mode
    logits_ref[...] = lg

    h = jnp.maximum(jnp.dot(lg, w0_ref[...], preferred_element_type=jnp.float32), 0.0)
    lp = jnp.dot(h, w1_ref[...], preferred_element_type=jnp.float32)
    lp_ref[...] = lp

    mx = jnp.max(lp, axis=-1, keepdims=True)
    lse = mx + jnp.log(jnp.sum(jnp.exp(lp - mx), axis=-1, keepdims=True))
    ce = -jnp.sum(gt_ref[...] * (lp - lse), axis=-1, keepdims=True)
    m = m_ref[0]

    @pl.when(i == 0)
    def _():
        acc_ref[0] = 0.0
        acc_ref[1] = 0.0

    acc_ref[0] += jnp.dot(m, ce, preferred_element_type=jnp.float32)[0, 0]
    acc_ref[1] += jnp.sum(m)

    @pl.when(i == pl.num_programs(0) - 1)
    def _():
        w0 = w0_ref[...]
        w1 = w1_ref[...]
        l2 = 0.5 * (jnp.sum(w0 * w0) + jnp.sum(w1 * w1))
        loss_ref[...] = jnp.reshape(acc_ref[0] / acc_ref[1] + 0.0002 * l2, (1, 1))


def _dense_pass(inputs, ground_truth, masks_f, w_pad, b2, w0_pad, out_w1):
    grid = (_N // _R,)
    return pl.pallas_call(
        _dense_body,
        grid=grid,
        in_specs=[
            pl.BlockSpec((_R, _F // 2), lambda i: (i, 0)),
            pl.BlockSpec((_R, _F // 2), lambda i: (i, 1)),
            pl.BlockSpec((_R, _O), lambda i: (i, 0)),
            pl.BlockSpec((1, 1, _R), lambda i: (i, 0, 0)),
            pl.BlockSpec((_F, _EP), lambda i: (0, 0)),
            pl.BlockSpec((1, 1), lambda i: (0, 0)),
            pl.BlockSpec((_EP, _H), lambda i: (0, 0)),
            pl.BlockSpec((_H, _O), lambda i: (0, 0)),
        ],
        out_specs=[
            pl.BlockSpec((_R, _EP), lambda i: (i, 0)),
            pl.BlockSpec((_R, _O), lambda i: (i, 0)),
            pl.BlockSpec((1, 1), lambda i: (0, 0)),
        ],
        out_shape=[
            jax.ShapeDtypeStruct((_N, _EP), jnp.float32),
            jax.ShapeDtypeStruct((_N, _O), jnp.float32),
            jax.ShapeDtypeStruct((1, 1), jnp.float32),
        ],
        scratch_shapes=[pltpu.SMEM((2,), jnp.float32)],
        compiler_params=pltpu.CompilerParams(
            dimension_semantics=("arbitrary",)),
    )(inputs, inputs, ground_truth, masks_f, w_pad, b2, w0_pad, out_w1)


_NUM_IDS = _B * (2 + _NEG)
_CHUNK = 128
_EPG = 2 + _NEG
_DOT = 128


def _sc_gather_dots(table, ids):
    info = plsc.get_sparse_core_info()
    nw = info.num_cores * info.num_subcores
    per_w = _NUM_IDS // nw
    e_per_w = per_w // _EPG
    n_ch = per_w // _CHUNK
    mesh = plsc.VectorSubcoreMesh(core_axis_name="c", subcore_axis_name="s")

    @functools.partial(
        pl.kernel,
        mesh=mesh,
        out_type=jax.ShapeDtypeStruct((_B, _DOT), jnp.float32),
        scratch_types=[
            pltpu.VMEM((per_w,), jnp.int32),
            pltpu.VMEM((per_w, _EP), jnp.float32),
            pltpu.VMEM((e_per_w, _DOT), jnp.float32),
            pltpu.SemaphoreType.DMA,
        ],
    )
    def k(table_hbm, idx_hbm, out_hbm, idx_v, rows_v, dots_v, sem):
        wid = lax.axis_index("s") * info.num_cores + lax.axis_index("c")
        base = wid * per_w
        pltpu.sync_copy(idx_hbm.at[pl.ds(base, per_w)], idx_v)
        descs = []
        for c in range(n_ch):
            descs.append(pltpu.async_copy(
                table_hbm.at[idx_v.at[pl.ds(c * _CHUNK, _CHUNK)]],
                rows_v.at[pl.ds(c * _CHUNK, _CHUNK)],
                sem))

        def edge(e, carry):
            r0 = e * _EPG
            hi = [rows_v[r0, pl.ds(c * 16, 16)] for c in range(4)]
            for p in range(1 + _NEG):
                acc = hi[0] * rows_v[r0 + 1 + p, pl.ds(0, 16)]
                for c in range(1, 4):
                    acc += hi[c] * rows_v[r0 + 1 + p, pl.ds(c * 16, 16)]
                dots_v[e, pl.ds(p * 16, 16)] = acc
            return carry

        half_ch = n_ch // 2
        for d in descs[:half_ch]:
            d.wait()
        lax.fori_loop(0, e_per_w // 2, edge, 0)
        for d in descs[half_ch:]:
            d.wait()
        lax.fori_loop(e_per_w // 2, e_per_w, edge, 0)
        pltpu.sync_copy(dots_v, out_hbm.at[pl.ds(wid * e_per_w, e_per_w)])

    return k(table, ids)


def _struct_body(d_ref, out_ref):
    lane = lax.broadcasted_iota(jnp.int32, (80, 8), 0)
    grp = lax.broadcasted_iota(jnp.int32, (80, 8), 1)
    sel = (lane // 16) == grp
    s = jnp.where(sel & (grp == 0), -1.0, jnp.where(sel & (grp < 5), 1.0, 0.0))
    x = jnp.dot(d_ref[:, 0:80], s, preferred_element_type=jnp.float32)
    sp = jnp.log1p(jnp.exp(x))
    wv = jnp.where(grp[0:1, :] == 0, 1.0, jnp.where(grp[0:1, :] < 5, 1.0 / _NEG, 0.0))
    total = jnp.sum(sp * wv) / _B
    out_ref[...] = jnp.reshape(total, (1, 1))


def _struct_tail(dots):
    return pl.pallas_call(
        _struct_body,
        out_shape=jax.ShapeDtypeStruct((1, 1), jnp.float32),
    )(dots)


def kernel(inputs, ground_truth, masks, xi_id, xj_id, neg_xj_id, w, b, out_w0, out_w1):
    w_pad = jnp.pad(w, ((0, 0), (0, _EP - _E)))
    w0_pad = jnp.pad(out_w0, ((0, _EP - _E), (0, 0)))
    b2 = jnp.reshape(b, (1, 1))
    masks_f = masks.astype(jnp.float32).reshape(_N // _R, 1, _R)

    logits, label_pred, label_loss = _dense_pass(
        inputs, ground_truth, masks_f, w_pad, b2, w0_pad, out_w1)

    ids = jnp.concatenate([xi_id, xj_id, neg_xj_id], axis=1).reshape(-1).astype(jnp.int32)

    dots = _sc_gather_dots(logits, ids)
    struct = _struct_tail(dots)

    return struct[0, 0], label_loss[0, 0], label_pred


# speedup vs baseline: 1.0129x; 1.0004x over previous
import functools

import jax
import jax.numpy as jnp
from jax import lax
from jax.experimental import pallas as pl
from jax.experimental.pallas import tpu as pltpu
from jax.experimental.pallas import tpu_sc as plsc

_N = 100000
_F = 512
_E = 51
_EP = 128
_H = 128
_O = 64
_B = 4096
_NEG = 4
_R = 5000


def _dense_body(x1_ref, x2_ref, gt_ref, m_ref, w_ref, b_ref,
                w0_ref, w1_ref, logits_ref, lp_ref, loss_ref, acc_ref):
    i = pl.program_id(0)

    y = (jnp.dot(x1_ref[...], w_ref[0:_F // 2], preferred_element_type=jnp.float32)
         + jnp.dot(x2_ref[...], w_ref[_F // 2:_F], preferred_element_type=jnp.float32)
         + b_ref[...])
    y = jnp.maximum(y, 0.2 * y)
    col = lax.broadcasted_iota(jnp.int32, y.shape, 1)
    y = jnp.where(col < _E, y, 0.0)
    mode = jnp.sqrt(jnp.sum(y * y, axis=-1, keepdims=True))
    lg = y / mode
    logits_ref[...] = lg

    h = jnp.maximum(jnp.dot(lg, w0_ref[...], preferred_element_type=jnp.float32), 0.0)
    lp = jnp.dot(h, w1_ref[...], preferred_element_type=jnp.float32)
    lp_ref[...] = lp

    mx = jnp.max(lp, axis=-1, keepdims=True)
    lse = mx + jnp.log(jnp.sum(jnp.exp(lp - mx), axis=-1, keepdims=True))
    ce = -jnp.sum(gt_ref[...] * (lp - lse), axis=-1, keepdims=True)
    m = m_ref[0]

    @pl.when(i == 0)
    def _():
        acc_ref[0] = 0.0
        acc_ref[1] = 0.0

    acc_ref[0] += jnp.dot(m, ce, preferred_element_type=jnp.float32)[0, 0]
    acc_ref[1] += jnp.sum(m)

    @pl.when(i == pl.num_programs(0) - 1)
    def _():
        w0 = w0_ref[...]
        w1 = w1_ref[...]
        l2 = 0.5 * (jnp.sum(w0 * w0) + jnp.sum(w1 * w1))
        loss_ref[...] = jnp.reshape(acc_ref[0] / acc_ref[1] + 0.0002 * l2, (1, 1))


def _dense_pass(inputs, ground_truth, masks_f, w_pad, b2, w0_pad, out_w1):
    grid = (_N // _R,)
    return pl.pallas_call(
        _dense_body,
        grid=grid,
        in_specs=[
            pl.BlockSpec((_R, _F // 2), lambda i: (i, 0)),
            pl.BlockSpec((_R, _F // 2), lambda i: (i, 1)),
            pl.BlockSpec((_R, _O), lambda i: (i, 0)),
            pl.BlockSpec((1, 1, _R), lambda i: (i, 0, 0)),
            pl.BlockSpec((_F, _EP), lambda i: (0, 0)),
            pl.BlockSpec((1, 1), lambda i: (0, 0)),
            pl.BlockSpec((_EP, _H), lambda i: (0, 0)),
            pl.BlockSpec((_H, _O), lambda i: (0, 0)),
        ],
        out_specs=[
            pl.BlockSpec((_R, _EP), lambda i: (i, 0)),
            pl.BlockSpec((_R, _O), lambda i: (i, 0)),
            pl.BlockSpec((1, 1), lambda i: (0, 0)),
        ],
        out_shape=[
            jax.ShapeDtypeStruct((_N, _EP), jnp.float32),
            jax.ShapeDtypeStruct((_N, _O), jnp.float32),
            jax.ShapeDtypeStruct((1, 1), jnp.float32),
        ],
        scratch_shapes=[pltpu.SMEM((2,), jnp.float32)],
        compiler_params=pltpu.CompilerParams(
            dimension_semantics=("arbitrary",)),
    )(inputs, inputs, ground_truth, masks_f, w_pad, b2, w0_pad, out_w1)


_NUM_IDS = _B * (2 + _NEG)
_CHUNK = 128
_EPG = 2 + _NEG
_DOT = 128


def _sc_gather_dots(table, ids):
    info = plsc.get_sparse_core_info()
    nw = info.num_cores * info.num_subcores
    per_w = _NUM_IDS // nw
    e_per_w = per_w // _EPG
    n_ch = per_w // _CHUNK
    mesh = plsc.VectorSubcoreMesh(core_axis_name="c", subcore_axis_name="s")

    @functools.partial(
        pl.kernel,
        mesh=mesh,
        out_type=jax.ShapeDtypeStruct((_B, _DOT), jnp.float32),
        scratch_types=[
            pltpu.VMEM((per_w,), jnp.int32),
            pltpu.VMEM((per_w, _EP), jnp.float32),
            pltpu.VMEM((e_per_w, _DOT), jnp.float32),
            pltpu.SemaphoreType.DMA,
        ],
    )
    def k(table_hbm, idx_hbm, out_hbm, idx_v, rows_v, dots_v, sem):
        wid = lax.axis_index("s") * info.num_cores + lax.axis_index("c")
        base = wid * per_w
        pltpu.sync_copy(idx_hbm.at[pl.ds(base, per_w)], idx_v)
        descs = []
        for c in range(n_ch):
            descs.append(pltpu.async_copy(
                table_hbm.at[idx_v.at[pl.ds(c * _CHUNK, _CHUNK)]],
                rows_v.at[pl.ds(c * _CHUNK, _CHUNK)],
                sem))

        def edge(e, carry):
            r0 = e * _EPG
            hi = [rows_v[r0, pl.ds(c * 16, 16)] for c in range(4)]
            for p in range(1 + _NEG):
                acc = hi[0] * rows_v[r0 + 1 + p, pl.ds(0, 16)]
                for c in range(1, 4):
                    acc += hi[c] * rows_v[r0 + 1 + p, pl.ds(c * 16, 16)]
                dots_v[e, pl.ds(p * 16, 16)] = acc
            return carry

        half_ch = n_ch // 2
        for d in descs[:half_ch]:
            d.wait()
        lax.fori_loop(0, e_per_w // 2, edge, 0, unroll=2)
        for d in descs[half_ch:]:
            d.wait()
        lax.fori_loop(e_per_w // 2, e_per_w, edge, 0, unroll=2)
        pltpu.sync_copy(dots_v, out_hbm.at[pl.ds(wid * e_per_w, e_per_w)])

    return k(table, ids)


def _struct_body(d_ref, out_ref):
    lane = lax.broadcasted_iota(jnp.int32, (80, 8), 0)
    grp = lax.broadcasted_iota(jnp.int32, (80, 8), 1)
    sel = (lane // 16) == grp
    s = jnp.where(sel & (grp == 0), -1.0, jnp.where(sel & (grp < 5), 1.0, 0.0))
    x = jnp.dot(d_ref[:, 0:80], s, preferred_element_type=jnp.float32)
    sp = jnp.log1p(jnp.exp(x))
    wv = jnp.where(grp[0:1, :] == 0, 1.0, jnp.where(grp[0:1, :] < 5, 1.0 / _NEG, 0.0))
    total = jnp.sum(sp * wv) / _B
    out_ref[...] = jnp.reshape(total, (1, 1))


def _struct_tail(dots):
    return pl.pallas_call(
        _struct_body,
        out_shape=jax.ShapeDtypeStruct((1, 1), jnp.float32),
    )(dots)


def kernel(inputs, ground_truth, masks, xi_id, xj_id, neg_xj_id, w, b, out_w0, out_w1):
    w_pad = jnp.pad(w, ((0, 0), (0, _EP - _E)))
    w0_pad = jnp.pad(out_w0, ((0, _EP - _E), (0, 0)))
    b2 = jnp.reshape(b, (1, 1))
    masks_f = masks.astype(jnp.float32).reshape(_N // _R, 1, _R)

    logits, label_pred, label_loss = _dense_pass(
        inputs, ground_truth, masks_f, w_pad, b2, w0_pad, out_w1)

    ids = jnp.concatenate([xi_id, xj_id, neg_xj_id], axis=1).reshape(-1).astype(jnp.int32)

    dots = _sc_gather_dots(logits, ids)
    struct = _struct_tail(dots)

    return struct[0, 0], label_loss[0, 0], label_pred
